# baseline (speedup 1.0000x reference)
import functools

import jax
import jax.numpy as jnp
from jax import lax
from jax.experimental import pallas as pl
from jax.experimental.pallas import tpu as pltpu
from jax.experimental.pallas import tpu_sc as plsc

N_N = 10000
N_E = 320000
D_N = 128
D_E = 16
G_D = 64
H = 64
H3 = 3 * H

NC = 2
NS = 16
NW = NC * NS

EPT_G = N_E // NW
EPT_S = N_E // NW

G_SUB = 80
G_NSUB = 5
G_ROWS = G_SUB * G_NSUB
G_NGRP = EPT_G // G_ROWS

S_SUB = 100
S_NSUB = 10
S_ROWS = S_SUB * S_NSUB
S_NGRP = EPT_S // S_ROWS

N_PAD = 10240
RPT = N_PAD // NS


def _scg_body(x_hbm, src_hbm, gx_out, src_v, rows_v, gsem, wsem):
    c = lax.axis_index("c")
    s = lax.axis_index("s")
    wid = c * NS + s
    base0 = wid * EPT_G
    pltpu.sync_copy(src_hbm.at[wid], src_v)

    def grp_body(g, carry):
        slot = lax.rem(g, 2)
        @pl.when(g >= 2)
        def _():
            pltpu.make_async_copy(
                gx_out.at[pl.ds(0, G_ROWS)], rows_v.at[0], wsem).wait()

        descs = []
        for k in range(G_NSUB):
            descs.append(pltpu.async_copy(
                x_hbm.at[src_v.at[g, k]],
                rows_v.at[slot, pl.ds(k * G_SUB, G_SUB)], gsem))
        for d in descs:
            d.wait()
        base = pl.multiple_of(base0 + g * G_ROWS, 8)
        pltpu.async_copy(rows_v.at[slot], gx_out.at[pl.ds(base, G_ROWS)], wsem)
        return carry

    lax.fori_loop(0, G_NGRP, grp_body, 0)
    pltpu.make_async_copy(gx_out.at[pl.ds(0, G_ROWS)], rows_v.at[0], wsem).wait()
    pltpu.make_async_copy(gx_out.at[pl.ds(0, G_ROWS)], rows_v.at[0], wsem).wait()


def _sc_gather(x, src3):
    mesh = plsc.VectorSubcoreMesh(core_axis_name="c", subcore_axis_name="s",
                                  num_cores=NC, num_subcores=NS)
    fn = pl.kernel(
        _scg_body,
        out_type=jax.ShapeDtypeStruct((N_E, D_N), jnp.bfloat16),
        mesh=mesh,
        scratch_types=[
            pltpu.VMEM((G_NGRP, G_NSUB, G_SUB), jnp.int32),
            pltpu.VMEM((2, G_ROWS, D_N), jnp.bfloat16),
            pltpu.SemaphoreType.DMA,
            pltpu.SemaphoreType.DMA,
        ],
        compiler_params=pltpu.CompilerParams(use_tc_tiling_on_sc=False),
    )
    return fn(x, src3)


def _tce_body(gx_ref, e_ref, w2et_ref, w1et_ref, g_ref, wget_ref, be_ref,
              bne_ref, he_ref):
    const_e = jnp.dot(g_ref[...], wget_ref[...],
                      preferred_element_type=jnp.float32) + be_ref[...]
    gi = jnp.dot(gx_ref[...], w2et_ref[...],
                 preferred_element_type=jnp.float32)
    gi = gi + jnp.dot(e_ref[...], w1et_ref[...],
                      preferred_element_type=jnp.float32) + const_e
    i_r = gi[:, :H]
    i_z = gi[:, H:2 * H]
    i_n = gi[:, 2 * H:]
    r = 0.5 + 0.5 * jnp.tanh(0.5 * i_r)
    omz = 0.5 - 0.5 * jnp.tanh(0.5 * i_z)
    n = jnp.tanh(i_n + r * bne_ref[...])
    he_ref[...] = omz * n


def _tce(gx, e, w2et, w1et, g, wget, be, bne):
    blk = 4000
    grid = N_E // blk
    return pl.pallas_call(
        _tce_body,
        grid=(grid,),
        in_specs=[
            pl.BlockSpec((blk, D_N), lambda i: (i, 0)),
            pl.BlockSpec((blk, D_E), lambda i: (i, 0)),
            pl.BlockSpec((D_N, H3), lambda i: (0, 0)),
            pl.BlockSpec((D_E, H3), lambda i: (0, 0)),
            pl.BlockSpec((1, G_D), lambda i: (0, 0)),
            pl.BlockSpec((G_D, H3), lambda i: (0, 0)),
            pl.BlockSpec((1, H3), lambda i: (0, 0)),
            pl.BlockSpec((1, H), lambda i: (0, 0)),
        ],
        out_specs=pl.BlockSpec((blk, H), lambda i: (i, 0)),
        out_shape=jax.ShapeDtypeStruct((N_E, H), jnp.float32),
    )(gx, e, w2et, w1et, g, wget, be, bne)


def _scs_body(he_hbm, dst_hbm, zeros_hbm, agg_out,
              dst_v, h_v, agg_sh, lsem):
    c = lax.axis_index("c")
    s = lax.axis_index("s")
    wid = c * NS + s
    base0 = wid * EPT_S

    pltpu.sync_copy(zeros_hbm.at[pl.ds(s * RPT, RPT)],
                    agg_sh.at[pl.ds(s * RPT, RPT)])
    pltpu.sync_copy(dst_hbm.at[wid], dst_v)
    plsc.subcore_barrier()

    def grp_body(g, carry):
        base = pl.multiple_of(base0 + g * S_ROWS, 8)
        pltpu.async_copy(he_hbm.at[pl.ds(base, S_ROWS)], h_v, lsem).wait()
        for k in range(S_NSUB):
            pltpu.sync_copy(h_v.at[pl.ds(k * S_SUB, S_SUB)],
                            agg_sh.at[dst_v.at[g * S_NSUB + k]], add=True)
        return carry

    lax.fori_loop(0, S_NGRP, grp_body, 0)
    plsc.subcore_barrier()
    pltpu.sync_copy(agg_sh.at[pl.ds(s * RPT, RPT)],
                    agg_out.at[c, pl.ds(s * RPT, RPT)])


def _sc_scatter(he, dst3, zeros_n):
    mesh = plsc.VectorSubcoreMesh(core_axis_name="c", subcore_axis_name="s",
                                  num_cores=NC, num_subcores=NS)
    fn = pl.kernel(
        _scs_body,
        out_type=jax.ShapeDtypeStruct((NC, N_PAD, H), jnp.float32),
        mesh=mesh,
        scratch_types=[
            pltpu.VMEM((S_NGRP * S_NSUB, S_SUB), jnp.int32),
            pltpu.VMEM((S_ROWS, H), jnp.float32),
            pltpu.VMEM_SHARED((N_PAD, H), jnp.float32),
            pltpu.SemaphoreType.DMA,
        ],
        compiler_params=pltpu.CompilerParams(use_tc_tiling_on_sc=False),
    )
    return fn(he, dst3, zeros_n)


def _tc2_body(aggp_ref, x_ref, w1nt_ref, w2nt_ref, g_ref, wgnt_ref, bn_ref,
              bnn_ref, wgt_ref, bg_ref, hv_ref, gnew_ref, acc_ref):
    i = pl.program_id(0)
    agg = aggp_ref[0] + aggp_ref[1]
    const_n = jnp.dot(g_ref[...], wgnt_ref[...],
                      preferred_element_type=jnp.float32) + bn_ref[...]
    gin = jnp.dot(agg, w1nt_ref[...], preferred_element_type=jnp.float32) \
        + jnp.dot(x_ref[...], w2nt_ref[...], preferred_element_type=jnp.float32) \
        + const_n
    i_r = gin[:, :H]
    i_z = gin[:, H:2 * H]
    i_n = gin[:, 2 * H:]
    r = 0.5 + 0.5 * jnp.tanh(0.5 * i_r)
    omz = 0.5 - 0.5 * jnp.tanh(0.5 * i_z)
    n = jnp.tanh(i_n + r * bnn_ref[...])
    hv = omz * n
    hv_ref[...] = hv
    part = jnp.concatenate([jnp.sum(agg, axis=0, keepdims=True),
                            jnp.sum(hv, axis=0, keepdims=True)], axis=1)

    @pl.when(i == 0)
    def _():
        acc_ref[...] = part

    @pl.when(i > 0)
    def _():
        acc_ref[...] = acc_ref[...] + part

    @pl.when(i == pl.num_programs(0) - 1)
    def _():
        mean_he = acc_ref[0:1, :H] * (1.0 / N_E)
        mean_hv = acc_ref[0:1, H:] * (1.0 / N_N)
        g_in = jnp.concatenate([mean_he, mean_hv, g_ref[...]], axis=1)
        gnew_ref[...] = jnp.maximum(
            jnp.dot(g_in, wgt_ref[...], preferred_element_type=jnp.float32)
            + bg_ref[...], 0.0)


def _tc2(aggp, x, w1nt, w2nt, g, wgnt, bn, bnn, wgt, bg):
    blk = 1000
    grid = N_N // blk
    return pl.pallas_call(
        _tc2_body,
        grid=(grid,),
        in_specs=[
            pl.BlockSpec((NC, blk, H), lambda i: (0, i, 0)),
            pl.BlockSpec((blk, D_N), lambda i: (i, 0)),
            pl.BlockSpec((H, H3), lambda i: (0, 0)),
            pl.BlockSpec((D_N, H3), lambda i: (0, 0)),
            pl.BlockSpec((1, G_D), lambda i: (0, 0)),
            pl.BlockSpec((G_D, H3), lambda i: (0, 0)),
            pl.BlockSpec((1, H3), lambda i: (0, 0)),
            pl.BlockSpec((1, H), lambda i: (0, 0)),
            pl.BlockSpec((H3, G_D), lambda i: (0, 0)),
            pl.BlockSpec((1, G_D), lambda i: (0, 0)),
        ],
        out_specs=[
            pl.BlockSpec((blk, H), lambda i: (i, 0)),
            pl.BlockSpec((1, G_D), lambda i: (0, 0)),
        ],
        out_shape=[
            jax.ShapeDtypeStruct((N_N, H), jnp.float32),
            jax.ShapeDtypeStruct((1, G_D), jnp.float32),
        ],
        scratch_shapes=[pltpu.VMEM((1, 2 * H), jnp.float32)],
    )(aggp, x, w1nt, w2nt, g, wgnt, bn, bnn, wgt, bg)


def kernel(x, e, edge_index, global_attr, W_ih_e, W_hh_e, b_ih_e, b_hh_e,
           W_ih_n, W_hh_n, b_ih_n, b_hh_n, W_g, b_g):
    w1et = W_ih_e[:, :D_E].T
    w2et = W_ih_e[:, D_E:D_E + D_N].T
    wget = W_ih_e[:, D_E + D_N:].T
    be = (b_ih_e + jnp.concatenate([b_hh_e[:2 * H],
                                    jnp.zeros((H,), jnp.float32)]))[None]
    bne = b_hh_e[2 * H:][None]
    w1nt = W_ih_n[:, :H].T
    w2nt = W_ih_n[:, H:H + D_N].T
    wgnt = W_ih_n[:, H + D_N:].T
    bn = (b_ih_n + jnp.concatenate([b_hh_n[:2 * H],
                                    jnp.zeros((H,), jnp.float32)]))[None]
    bnn = b_hh_n[2 * H:][None]
    wgt = W_g.T
    bg = b_g[None]
    src3 = edge_index[0].reshape(NW, G_NGRP, G_NSUB, G_SUB)
    dst3 = edge_index[1].reshape(NW, S_NGRP * S_NSUB, S_SUB)
    zeros_n = jnp.zeros((N_PAD, H), jnp.float32)
    xb = x.astype(jnp.bfloat16)
    w2et_b = w2et.astype(jnp.bfloat16)

    gx = _sc_gather(xb, src3)
    h_e = _tce(gx, e, w2et_b, w1et, global_attr, wget, be, bne)
    aggp = _sc_scatter(h_e, dst3, zeros_n)
    h_v, g_new = _tc2(aggp, x, w1nt, w2nt, global_attr, wgnt, bn, bnn,
                      wgt, bg)
    return (h_e, h_v, g_new)


# speedup vs baseline: 1.4330x; 1.4330x over previous
import functools

import jax
import jax.numpy as jnp
from jax import lax
from jax.experimental import pallas as pl
from jax.experimental.pallas import tpu as pltpu
from jax.experimental.pallas import tpu_sc as plsc

N_N = 10000
N_E = 320000
D_N = 128
D_E = 16
G_D = 64
H = 64
H3 = 3 * H

NC = 2
NS = 16
NW = NC * NS

EPT_G = N_E // NW
EPT_S = N_E // NW

G_SUB = 80
G_NSUB = 5
G_ROWS = G_SUB * G_NSUB
G_NGRP = EPT_G // G_ROWS

S_SUB = 100
S_NSUB = 10
S_ROWS = S_SUB * S_NSUB
S_NGRP = EPT_S // S_ROWS

N_PAD = 10240
RPT = N_PAD // NS


def _scg_body(x_hbm, src_hbm, gx_out, src_v, rows_v, gsem, wsem):
    c = lax.axis_index("c")
    s = lax.axis_index("s")
    wid = c * NS + s
    base0 = wid * EPT_G
    pltpu.sync_copy(src_hbm.at[wid], src_v)

    def grp_body(g, carry):
        slot = lax.rem(g, 2)
        @pl.when(g >= 2)
        def _():
            pltpu.make_async_copy(
                gx_out.at[pl.ds(0, G_ROWS)], rows_v.at[0], wsem).wait()

        descs = []
        for k in range(G_NSUB):
            descs.append(pltpu.async_copy(
                x_hbm.at[src_v.at[g, k]],
                rows_v.at[slot, pl.ds(k * G_SUB, G_SUB)], gsem))
        for d in descs:
            d.wait()
        base = pl.multiple_of(base0 + g * G_ROWS, 8)
        pltpu.async_copy(rows_v.at[slot], gx_out.at[pl.ds(base, G_ROWS)], wsem)
        return carry

    lax.fori_loop(0, G_NGRP, grp_body, 0)
    pltpu.make_async_copy(gx_out.at[pl.ds(0, G_ROWS)], rows_v.at[0], wsem).wait()
    pltpu.make_async_copy(gx_out.at[pl.ds(0, G_ROWS)], rows_v.at[0], wsem).wait()


def _sc_gather(x, src3):
    mesh = plsc.VectorSubcoreMesh(core_axis_name="c", subcore_axis_name="s",
                                  num_cores=NC, num_subcores=NS)
    fn = pl.kernel(
        _scg_body,
        out_type=jax.ShapeDtypeStruct((N_E, D_N), jnp.float32),
        mesh=mesh,
        scratch_types=[
            pltpu.VMEM((G_NGRP, G_NSUB, G_SUB), jnp.int32),
            pltpu.VMEM((2, G_ROWS, D_N), jnp.float32),
            pltpu.SemaphoreType.DMA,
            pltpu.SemaphoreType.DMA,
        ],
        compiler_params=pltpu.CompilerParams(use_tc_tiling_on_sc=False),
    )
    return fn(x, src3)


def _tce_body(gx_ref, e_ref, w2et_ref, w1et_ref, g_ref, wget_ref, be_ref,
              bne_ref, he_ref):
    const_e = jnp.dot(g_ref[...], wget_ref[...],
                      preferred_element_type=jnp.float32) + be_ref[...]
    gi = jnp.dot(gx_ref[...], w2et_ref[...],
                 preferred_element_type=jnp.float32)
    gi = gi + jnp.dot(e_ref[...], w1et_ref[...],
                      preferred_element_type=jnp.float32) + const_e
    i_r = gi[:, :H]
    i_z = gi[:, H:2 * H]
    i_n = gi[:, 2 * H:]
    r = 0.5 + 0.5 * jnp.tanh(0.5 * i_r)
    omz = 0.5 - 0.5 * jnp.tanh(0.5 * i_z)
    n = jnp.tanh(i_n + r * bne_ref[...])
    he_ref[...] = omz * n


def _tce(gx, e, w2et, w1et, g, wget, be, bne):
    blk = 4000
    grid = N_E // blk
    return pl.pallas_call(
        _tce_body,
        grid=(grid,),
        in_specs=[
            pl.BlockSpec((blk, D_N), lambda i: (i, 0)),
            pl.BlockSpec((blk, D_E), lambda i: (i, 0)),
            pl.BlockSpec((D_N, H3), lambda i: (0, 0)),
            pl.BlockSpec((D_E, H3), lambda i: (0, 0)),
            pl.BlockSpec((1, G_D), lambda i: (0, 0)),
            pl.BlockSpec((G_D, H3), lambda i: (0, 0)),
            pl.BlockSpec((1, H3), lambda i: (0, 0)),
            pl.BlockSpec((1, H), lambda i: (0, 0)),
        ],
        out_specs=pl.BlockSpec((blk, H), lambda i: (i, 0)),
        out_shape=jax.ShapeDtypeStruct((N_E, H), jnp.float32),
    )(gx, e, w2et, w1et, g, wget, be, bne)


def _scs_body(he_hbm, dst_hbm, zeros_hbm, agg_out,
              dst_v, h_v, agg_sh, lsem):
    c = lax.axis_index("c")
    s = lax.axis_index("s")
    wid = c * NS + s
    base0 = wid * EPT_S

    pltpu.sync_copy(zeros_hbm.at[pl.ds(s * RPT, RPT)],
                    agg_sh.at[pl.ds(s * RPT, RPT)])
    pltpu.sync_copy(dst_hbm.at[wid], dst_v)
    plsc.subcore_barrier()

    def grp_body(g, carry):
        base = pl.multiple_of(base0 + g * S_ROWS, 8)
        pltpu.async_copy(he_hbm.at[pl.ds(base, S_ROWS)], h_v, lsem).wait()
        for k in range(S_NSUB):
            pltpu.sync_copy(h_v.at[pl.ds(k * S_SUB, S_SUB)],
                            agg_sh.at[dst_v.at[g * S_NSUB + k]], add=True)
        return carry

    lax.fori_loop(0, S_NGRP, grp_body, 0)
    plsc.subcore_barrier()
    pltpu.sync_copy(agg_sh.at[pl.ds(s * RPT, RPT)],
                    agg_out.at[c, pl.ds(s * RPT, RPT)])


def _sc_scatter(he, dst3, zeros_n):
    mesh = plsc.VectorSubcoreMesh(core_axis_name="c", subcore_axis_name="s",
                                  num_cores=NC, num_subcores=NS)
    fn = pl.kernel(
        _scs_body,
        out_type=jax.ShapeDtypeStruct((NC, N_PAD, H), jnp.float32),
        mesh=mesh,
        scratch_types=[
            pltpu.VMEM((S_NGRP * S_NSUB, S_SUB), jnp.int32),
            pltpu.VMEM((S_ROWS, H), jnp.float32),
            pltpu.VMEM_SHARED((N_PAD, H), jnp.float32),
            pltpu.SemaphoreType.DMA,
        ],
        compiler_params=pltpu.CompilerParams(use_tc_tiling_on_sc=False),
    )
    return fn(he, dst3, zeros_n)


def _tc2_body(aggp_ref, x_ref, w1nt_ref, w2nt_ref, g_ref, wgnt_ref, bn_ref,
              bnn_ref, wgt_ref, bg_ref, hv_ref, gnew_ref, acc_ref):
    i = pl.program_id(0)
    agg = aggp_ref[0] + aggp_ref[1]
    const_n = jnp.dot(g_ref[...], wgnt_ref[...],
                      preferred_element_type=jnp.float32) + bn_ref[...]
    gin = jnp.dot(agg, w1nt_ref[...], preferred_element_type=jnp.float32) \
        + jnp.dot(x_ref[...], w2nt_ref[...], preferred_element_type=jnp.float32) \
        + const_n
    i_r = gin[:, :H]
    i_z = gin[:, H:2 * H]
    i_n = gin[:, 2 * H:]
    r = 0.5 + 0.5 * jnp.tanh(0.5 * i_r)
    omz = 0.5 - 0.5 * jnp.tanh(0.5 * i_z)
    n = jnp.tanh(i_n + r * bnn_ref[...])
    hv = omz * n
    hv_ref[...] = hv
    part = jnp.concatenate([jnp.sum(agg, axis=0, keepdims=True),
                            jnp.sum(hv, axis=0, keepdims=True)], axis=1)

    @pl.when(i == 0)
    def _():
        acc_ref[...] = part

    @pl.when(i > 0)
    def _():
        acc_ref[...] = acc_ref[...] + part

    @pl.when(i == pl.num_programs(0) - 1)
    def _():
        mean_he = acc_ref[0:1, :H] * (1.0 / N_E)
        mean_hv = acc_ref[0:1, H:] * (1.0 / N_N)
        g_in = jnp.concatenate([mean_he, mean_hv, g_ref[...]], axis=1)
        gnew_ref[...] = jnp.maximum(
            jnp.dot(g_in, wgt_ref[...], preferred_element_type=jnp.float32)
            + bg_ref[...], 0.0)


def _tc2(aggp, x, w1nt, w2nt, g, wgnt, bn, bnn, wgt, bg):
    blk = 1000
    grid = N_N // blk
    return pl.pallas_call(
        _tc2_body,
        grid=(grid,),
        in_specs=[
            pl.BlockSpec((NC, blk, H), lambda i: (0, i, 0)),
            pl.BlockSpec((blk, D_N), lambda i: (i, 0)),
            pl.BlockSpec((H, H3), lambda i: (0, 0)),
            pl.BlockSpec((D_N, H3), lambda i: (0, 0)),
            pl.BlockSpec((1, G_D), lambda i: (0, 0)),
            pl.BlockSpec((G_D, H3), lambda i: (0, 0)),
            pl.BlockSpec((1, H3), lambda i: (0, 0)),
            pl.BlockSpec((1, H), lambda i: (0, 0)),
            pl.BlockSpec((H3, G_D), lambda i: (0, 0)),
            pl.BlockSpec((1, G_D), lambda i: (0, 0)),
        ],
        out_specs=[
            pl.BlockSpec((blk, H), lambda i: (i, 0)),
            pl.BlockSpec((1, G_D), lambda i: (0, 0)),
        ],
        out_shape=[
            jax.ShapeDtypeStruct((N_N, H), jnp.float32),
            jax.ShapeDtypeStruct((1, G_D), jnp.float32),
        ],
        scratch_shapes=[pltpu.VMEM((1, 2 * H), jnp.float32)],
    )(aggp, x, w1nt, w2nt, g, wgnt, bn, bnn, wgt, bg)


def kernel(x, e, edge_index, global_attr, W_ih_e, W_hh_e, b_ih_e, b_hh_e,
           W_ih_n, W_hh_n, b_ih_n, b_hh_n, W_g, b_g):
    w1et = W_ih_e[:, :D_E].T
    w2et = W_ih_e[:, D_E:D_E + D_N].T
    wget = W_ih_e[:, D_E + D_N:].T
    be = (b_ih_e + jnp.concatenate([b_hh_e[:2 * H],
                                    jnp.zeros((H,), jnp.float32)]))[None]
    bne = b_hh_e[2 * H:][None]
    w1nt = W_ih_n[:, :H].T
    w2nt = W_ih_n[:, H:H + D_N].T
    wgnt = W_ih_n[:, H + D_N:].T
    bn = (b_ih_n + jnp.concatenate([b_hh_n[:2 * H],
                                    jnp.zeros((H,), jnp.float32)]))[None]
    bnn = b_hh_n[2 * H:][None]
    wgt = W_g.T
    bg = b_g[None]
    src3 = edge_index[0].reshape(NW, G_NGRP, G_NSUB, G_SUB)
    dst3 = edge_index[1].reshape(NW, S_NGRP * S_NSUB, S_SUB)
    zeros_n = jnp.zeros((N_PAD, H), jnp.float32)

    gx = _sc_gather(x, src3)
    h_e = _tce(gx, e, w2et, w1et, global_attr, wget, be, bne)
    aggp = _sc_scatter(h_e, dst3, zeros_n)
    h_v, g_new = _tc2(aggp, x, w1nt, w2nt, global_attr, wgnt, bn, bnn,
                      wgt, bg)
    return (h_e, h_v, g_new)
